# baseline (speedup 1.0000x reference)
import functools

import jax
import jax.numpy as jnp
from jax import lax
from jax.experimental import pallas as pl
from jax.experimental.pallas import tpu as pltpu
from jax.experimental.pallas import tpu_sc as plsc

MODEL_DIM = 64
SEQ_LEN = 200
BATCH = 4096
SCALE = float(MODEL_DIM) ** 0.5

_info = plsc.get_sparse_core_info()
NC, NS = _info.num_cores, _info.num_subcores
NW = NC * NS
XROWS_W = BATCH // NW
CHUNK_X = 4
NCHUNK = XROWS_W // CHUNK_X
NBUF = 2
G_SPLITS = ((0, 128), (128, 72))

_mesh = plsc.VectorSubcoreMesh(core_axis_name="c", subcore_axis_name="s")


@functools.partial(
    pl.kernel,
    out_type=jax.ShapeDtypeStruct((BATCH, SEQ_LEN, MODEL_DIM), jnp.float32),
    mesh=_mesh,
    compiler_params=pltpu.CompilerParams(use_tc_tiling_on_sc=False),
    scratch_types=[
        pltpu.VMEM((NBUF, CHUNK_X, SEQ_LEN), jnp.int32),
        pltpu.VMEM((NBUF, CHUNK_X, SEQ_LEN, MODEL_DIM), jnp.float32),
        pltpu.SemaphoreType.DMA,
        pltpu.SemaphoreType.DMA,
        pltpu.SemaphoreType.DMA,
        pltpu.SemaphoreType.DMA,
    ],
)
def _sc_gather(x_hbm, table_hbm, out_hbm, idx_v, rows_v,
               g_sem0, g_sem1, o_sem0, o_sem1):
    wid = lax.axis_index("s") * NC + lax.axis_index("c")
    x_base = wid * XROWS_W
    g_sems = (g_sem0, g_sem1)
    o_sems = (o_sem0, o_sem1)

    gathers = [None] * NBUF
    out_cps = [None] * NBUF

    def start_chunk(c):
        k = c % NBUF
        b = x_base + c * CHUNK_X
        pltpu.sync_copy(x_hbm.at[pl.ds(b, CHUNK_X), :], idx_v.at[k])
        cps = []
        for r in range(CHUNK_X):
            for (off, ln) in G_SPLITS:
                cps.append(pltpu.async_copy(
                    table_hbm.at[idx_v.at[k, r, pl.ds(off, ln)]],
                    rows_v.at[k, r, pl.ds(off, ln)],
                    g_sems[k],
                ))
        gathers[k] = cps

    def finish_chunk(c):
        k = c % NBUF
        for cp in gathers[k]:
            cp.wait()
        b = x_base + c * CHUNK_X
        out_cps[k] = pltpu.async_copy(
            rows_v.at[k], out_hbm.at[pl.ds(b, CHUNK_X)], o_sems[k])

    for c in range(NCHUNK):
        k = c % NBUF
        if out_cps[k] is not None:
            out_cps[k].wait()
            out_cps[k] = None
        start_chunk(c)
        if c >= 1:
            finish_chunk(c - 1)
    finish_chunk(NCHUNK - 1)
    for k in range(NBUF):
        if out_cps[k] is not None:
            out_cps[k].wait()


def kernel(x, embedding, positional_encoding):
    gathered = _sc_gather(x, embedding * SCALE)
    return gathered + positional_encoding


# speedup vs baseline: 1.8039x; 1.8039x over previous
import functools

import jax
import jax.numpy as jnp
from jax import lax
from jax.experimental import pallas as pl
from jax.experimental.pallas import tpu as pltpu
from jax.experimental.pallas import tpu_sc as plsc

INPUT_DIM = 1000000
MODEL_DIM = 64
SEQ_LEN = 200
BATCH = 4096
LANES = 16
D_VECS = MODEL_DIM // LANES
SCALE = float(MODEL_DIM) ** 0.5

_info = plsc.get_sparse_core_info()
NC, NS = _info.num_cores, _info.num_subcores
NW = NC * NS
XROWS_W = BATCH // NW
CHUNK_X = 2
NCHUNK = XROWS_W // CHUNK_X
NBUF = 2
G_SPLITS = ((0, 128), (128, 72))

_mesh = plsc.VectorSubcoreMesh(core_axis_name="c", subcore_axis_name="s")

_TR_COLS = 2048


def _transpose_body(tin_ref, tout_ref):
    t = tin_ref[...]
    tt = jnp.swapaxes(t, 0, 1) * SCALE
    tout_ref[...] = jnp.concatenate([tt, tt], axis=1)


def _stage_table(embedding):
    return pl.pallas_call(
        _transpose_body,
        grid=(pl.cdiv(INPUT_DIM, _TR_COLS),),
        in_specs=[pl.BlockSpec((MODEL_DIM, _TR_COLS), lambda i: (0, i))],
        out_specs=pl.BlockSpec((_TR_COLS, 2 * MODEL_DIM), lambda i: (i, 0)),
        out_shape=jax.ShapeDtypeStruct((INPUT_DIM, 2 * MODEL_DIM),
                                       jnp.float32),
    )(embedding.T)


@functools.partial(
    pl.kernel,
    out_type=jax.ShapeDtypeStruct((BATCH, SEQ_LEN, 2 * MODEL_DIM),
                                   jnp.float32),
    mesh=_mesh,
    scratch_types=[
        pltpu.VMEM((NBUF, CHUNK_X, SEQ_LEN), jnp.int32),
        pltpu.VMEM((NBUF, CHUNK_X, SEQ_LEN, 2 * MODEL_DIM), jnp.float32),
        pltpu.VMEM((SEQ_LEN, MODEL_DIM), jnp.float32),
        pltpu.SemaphoreType.DMA,
        pltpu.SemaphoreType.DMA,
        pltpu.SemaphoreType.DMA,
        pltpu.SemaphoreType.DMA,
    ],
)
def _sc_gather(x_hbm, table_hbm, pos_hbm, out_hbm, idx_v, rows_v, pos_v,
               g_sem0, g_sem1, o_sem0, o_sem1):
    wid = lax.axis_index("s") * NC + lax.axis_index("c")
    x_base = wid * XROWS_W
    g_sems = (g_sem0, g_sem1)
    o_sems = (o_sem0, o_sem1)

    pltpu.sync_copy(pos_hbm.at[0], pos_v)

    gathers = [None] * NBUF
    out_cps = [None] * NBUF

    def start_chunk(c):
        k = c % NBUF
        b = x_base + c * CHUNK_X
        pltpu.sync_copy(x_hbm.at[pl.ds(b, CHUNK_X), :], idx_v.at[k])
        cps = []
        for r in range(CHUNK_X):
            for (off, ln) in G_SPLITS:
                cps.append(pltpu.async_copy(
                    table_hbm.at[idx_v.at[k, r, pl.ds(off, ln)]],
                    rows_v.at[k, r, pl.ds(off, ln)],
                    g_sems[k],
                ))
        gathers[k] = cps

    def finish_chunk(c):
        k = c % NBUF
        for cp in gathers[k]:
            cp.wait()

        def body(s, carry):
            for d in range(D_VECS):
                pv = pos_v[s, pl.ds(d * LANES, LANES)]
                for r in range(CHUNK_X):
                    v = rows_v[k, r, s, pl.ds(d * LANES, LANES)]
                    rows_v[k, r, s, pl.ds(d * LANES, LANES)] = v + pv
            return carry

        lax.fori_loop(0, SEQ_LEN, body, 0)
        b = x_base + c * CHUNK_X
        out_cps[k] = pltpu.async_copy(
            rows_v.at[k], out_hbm.at[pl.ds(b, CHUNK_X)], o_sems[k])

    for c in range(NCHUNK):
        k = c % NBUF
        if out_cps[k] is not None:
            out_cps[k].wait()
            out_cps[k] = None
        start_chunk(c)
        if c >= 1:
            finish_chunk(c - 1)
    finish_chunk(NCHUNK - 1)
    for k in range(NBUF):
        if out_cps[k] is not None:
            out_cps[k].wait()


def kernel(x, embedding, positional_encoding):
    table128 = _stage_table(embedding)
    wide = _sc_gather(x, table128, positional_encoding)
    return wide[:, :, :MODEL_DIM]
